# Initial kernel scaffold; baseline (speedup 1.0000x reference)
import jax
import jax.numpy as jnp
from jax.experimental import pallas as pl


def kernel(x, adj, adj_weight, W1, b1, Wns, bns, W3, b3):
    raise NotImplementedError("write your pallas kernel here")


# speedup vs baseline: 4.2760x; 4.2760x over previous
import functools

import jax
import jax.numpy as jnp
from jax import lax
from jax.experimental import pallas as pl
from jax.experimental.pallas import tpu as pltpu
from jax.experimental.pallas import tpu_sc as plsc

N_NODES = 10000
N_EDGES = 160000
NFEAT = 256
HID = 16
NCLASS = 40
D = 2 * HID

NC, NS, L = 2, 16, 16
EW = 128
NR = 1280
ROWS_PER_T = NR // NS
CHUNK_ROWS = 8
N_CHUNKS = ROWS_PER_T // CHUNK_ROWS
NE_PAD = NR * EW
NP = 10240
TR = NP // NS


def _spmm_body(with_deg, S_hbm, src_hbm, dst_hbm, w_hbm, z2_hbm, z1_hbm,
               *rest):
    if with_deg:
        (agg_out, deg_out, src_v, dst_v, w_v, idxg, idxs, rows1,
         S_sh, agg_sh, deg_sh, sem) = rest
    else:
        (agg_out, src_v, dst_v, w_v, idxg, idxs, rows1,
         S_sh, agg_sh, sem) = rest

    c = lax.axis_index("c")
    s = lax.axis_index("s")

    pltpu.sync_copy(S_hbm.at[c, pl.ds(s * TR * HID, TR * HID)],
                    S_sh.at[pl.ds(s * TR * HID, TR * HID)])
    pltpu.sync_copy(z2_hbm.at[pl.ds(s * TR * HID, TR * HID)],
                    agg_sh.at[pl.ds(s * TR * HID, TR * HID)])
    if with_deg:
        @pl.when(c == 0)
        def _():
            pltpu.sync_copy(z1_hbm.at[pl.ds(s * TR, TR)],
                            deg_sh.at[pl.ds(s * TR, TR)])

    plsc.subcore_barrier()

    def chunk(i, carry):
        r0 = s * ROWS_PER_T + i * CHUNK_ROWS
        pltpu.sync_copy(src_hbm.at[pl.ds(r0, CHUNK_ROWS)], src_v)
        pltpu.sync_copy(dst_hbm.at[pl.ds(r0, CHUNK_ROWS)], dst_v)
        pltpu.sync_copy(w_hbm.at[pl.ds(r0, CHUNK_ROWS)], w_v)

        def row(j, carry2):
            def bld(g, carry3):
                s16 = src_v[j, pl.ds(g * L, L)] * HID
                d16 = dst_v[j, pl.ds(g * L, L)] * HID
                for f in range(HID):
                    idxg[pl.ds(g * L * HID + f * L, L)] = s16 + f
                    idxs[pl.ds(g * L * HID + f * L, L)] = d16 + f
                return carry3

            lax.fori_loop(0, EW // L, bld, 0)
            pltpu.async_copy(S_sh.at[idxg], rows1, sem).wait()

            def scale(g, carry3):
                w16 = w_v[j, pl.ds(g * L, L)]
                for f in range(HID):
                    o = g * L * HID + f * L
                    rows1[pl.ds(o, L)] = rows1[pl.ds(o, L)] * w16
                return carry3

            lax.fori_loop(0, EW // L, scale, 0)
            pltpu.sync_copy(rows1, agg_sh.at[idxs], add=True)
            if with_deg:
                @pl.when(c == 0)
                def _():
                    pltpu.sync_copy(w_v.at[j], deg_sh.at[dst_v.at[j]],
                                    add=True)
            return carry2

        lax.fori_loop(0, CHUNK_ROWS, row, 0)
        return carry

    lax.fori_loop(0, N_CHUNKS, chunk, 0)

    plsc.subcore_barrier()

    pltpu.sync_copy(agg_sh.at[pl.ds(s * TR * HID, TR * HID)],
                    agg_out.at[c, pl.ds(s * TR * HID, TR * HID)])
    if with_deg:
        @pl.when(c == 0)
        def _():
            pltpu.sync_copy(deg_sh.at[pl.ds(s * TR, TR)],
                            deg_out.at[pl.ds(s * TR, TR)])


def _make_spmm(with_deg):
    mesh = plsc.VectorSubcoreMesh(core_axis_name="c", subcore_axis_name="s",
                                  num_cores=NC, num_subcores=NS)
    out_type = [jax.ShapeDtypeStruct((NC, NP * HID), jnp.float32)]
    scratch = [
        pltpu.VMEM((CHUNK_ROWS, EW), jnp.int32),
        pltpu.VMEM((CHUNK_ROWS, EW), jnp.int32),
        pltpu.VMEM((CHUNK_ROWS, EW), jnp.float32),
        pltpu.VMEM((EW * HID,), jnp.int32),
        pltpu.VMEM((EW * HID,), jnp.int32),
        pltpu.VMEM((EW * HID,), jnp.float32),
    ]
    if with_deg:
        out_type.append(jax.ShapeDtypeStruct((NP,), jnp.float32))
    scratch.append(pltpu.VMEM_SHARED((NP * HID,), jnp.float32))
    scratch.append(pltpu.VMEM_SHARED((NP * HID,), jnp.float32))
    if with_deg:
        scratch.append(pltpu.VMEM_SHARED((NP,), jnp.float32))
    scratch.append(pltpu.SemaphoreType.DMA)

    return pl.kernel(functools.partial(_spmm_body, with_deg),
                     out_type=tuple(out_type), mesh=mesh,
                     scratch_types=tuple(scratch))


_spmm_deg = _make_spmm(True)
_spmm_plain = _make_spmm(False)


def _mm_body(x_ref, w_ref, o_ref):
    r = jnp.dot(x_ref[...], w_ref[...], preferred_element_type=jnp.float32)
    o_ref[0] = r[:, :HID]
    o_ref[1] = r[:, HID:]


def _tc_matmul(x, Wcat):
    return pl.pallas_call(
        _mm_body,
        grid=(10,),
        in_specs=[pl.BlockSpec((1000, NFEAT), lambda i: (i, 0)),
                  pl.BlockSpec((NFEAT, D), lambda i: (0, 0))],
        out_specs=pl.BlockSpec((NC, 1000, HID), lambda i: (0, i, 0)),
        out_shape=jax.ShapeDtypeStruct((NC, N_NODES, HID), jnp.float32),
    )(x, Wcat)


def _post_body(agg_ref, deg_ref, b1_ref, bns_ref, o_ref):
    d = deg_ref[...]
    o_ref[0] = jnp.maximum(agg_ref[0] + b1_ref[...], 0.0)
    o_ref[1] = jnp.maximum(agg_ref[1] / (d + 1e-6) + bns_ref[...], 0.0)


def _tc_post(agg, deg, b1, bns):
    return pl.pallas_call(
        _post_body,
        grid=(5,),
        in_specs=[pl.BlockSpec((NC, 2048, HID), lambda i: (0, i, 0)),
                  pl.BlockSpec((2048, 1), lambda i: (i, 0)),
                  pl.BlockSpec((1, HID), lambda i: (0, 0)),
                  pl.BlockSpec((1, HID), lambda i: (0, 0))],
        out_specs=pl.BlockSpec((NC, 2048, HID), lambda i: (0, i, 0)),
        out_shape=jax.ShapeDtypeStruct((NC, NP, HID), jnp.float32),
    )(agg, deg.reshape(NP, 1), b1.reshape(1, HID), bns.reshape(1, HID))


def _final_body(agg_ref, w_ref, b_ref, o_ref):
    a = jnp.concatenate([agg_ref[0], agg_ref[1]], axis=1)
    o = jnp.dot(a, w_ref[...], preferred_element_type=jnp.float32) + b_ref[...]
    m = jnp.max(o, axis=1, keepdims=True)
    lse = jnp.log(jnp.sum(jnp.exp(o - m), axis=1, keepdims=True)) + m
    o_ref[...] = o - lse


def _tc_final(agg2, W3, b3):
    return pl.pallas_call(
        _final_body,
        grid=(5,),
        in_specs=[pl.BlockSpec((NC, 2048, HID), lambda i: (0, i, 0)),
                  pl.BlockSpec((D, NCLASS), lambda i: (0, 0)),
                  pl.BlockSpec((1, NCLASS), lambda i: (0, 0))],
        out_specs=pl.BlockSpec((2048, NCLASS), lambda i: (i, 0)),
        out_shape=jax.ShapeDtypeStruct((NP, NCLASS), jnp.float32),
    )(agg2, W3, b3.reshape(1, NCLASS))


def kernel(x, adj, adj_weight, W1, b1, Wns, bns, W3, b3):
    src = adj[0]
    dst = adj[1]
    npad = NE_PAD - N_EDGES
    pidx = (jnp.arange(npad, dtype=jnp.int32) * 13) % N_NODES
    src_p = jnp.concatenate([src, pidx]).reshape(NR, EW)
    dst_p = jnp.concatenate([dst, pidx]).reshape(NR, EW)
    w_p = jnp.concatenate(
        [adj_weight, jnp.zeros((npad,), jnp.float32)]).reshape(NR, EW)
    z2 = jnp.zeros((NP * HID,), jnp.float32)
    z1 = jnp.zeros((NP,), jnp.float32)

    Wcat = jnp.concatenate([W1, Wns], axis=1)
    S2 = _tc_matmul(x, Wcat)
    S2 = jnp.concatenate(
        [S2, jnp.zeros((NC, NP - N_NODES, HID), jnp.float32)], axis=1)
    S2f = S2.reshape(NC, NP * HID)

    agg, deg = _spmm_deg(S2f, src_p, dst_p, w_p, z2, z1)
    h2 = _tc_post(agg.reshape(NC, NP, HID), deg, b1, bns)
    (agg2,) = _spmm_plain(h2.reshape(NC, NP * HID), src_p, dst_p, w_p, z2, z1)
    agg2 = agg2.reshape(NC, NP, HID)
    out = _tc_final(agg2, W3, b3)
    return out[:N_NODES]
